# baseline (speedup 1.0000x reference)
import functools

import jax
import jax.numpy as jnp
from jax.experimental import pallas as pl
from jax.experimental.pallas import tpu as pltpu

_MASK_RATIO = 0.75
_WINDOW = (16, 16, 16)


def _keep_kernel(nrow_ref, ncol_ref, keep_ref, *, num_windows, len_keep):
    n = nrow_ref[0]
    nc = ncol_ref[0]
    wp = jax.lax.broadcasted_iota(jnp.int32, (num_windows, num_windows), 0)
    wo = jax.lax.broadcasted_iota(jnp.int32, (num_windows, num_windows), 1)
    m = (nc < n) | ((nc == n) & (wp < wo))
    rank = jnp.sum(m.astype(jnp.int32), axis=0, keepdims=True)
    keep_ref[0] = (rank < len_keep).astype(jnp.float32)


def _slab_vis(keep_ref, *, nww, nwd, W, DC):
    b = pl.program_id(0)
    i = pl.program_id(1)
    base = i * (nww * nwd)
    jwin = jax.lax.broadcasted_iota(jnp.int32, (W, DC), 0) // (W // nww)
    kwin = jax.lax.broadcasted_iota(jnp.int32, (W, DC), 1) // (DC // nwd)
    vis = jnp.zeros((W, DC), jnp.float32)
    for j in range(nww):
        for k in range(nwd):
            kv = keep_ref[b, base + j * nwd + k]
            vis = jnp.where((jwin == j) & (kwin == k), kv, vis)
    return vis


def _maskout_kernel(keep_ref, mask_ref, *, nww, nwd, W, DC):
    vis = _slab_vis(keep_ref, nww=nww, nwd=nwd, W=W, DC=DC)
    mask_ref[...] = jnp.broadcast_to(1.0 - vis[None, None], mask_ref.shape)


def _xmask_kernel(keep_ref, x_ref, xm_ref, *, nww, nwd, ww, DC):
    b = pl.program_id(0)
    i = pl.program_id(1)
    j = pl.program_id(2)
    base = i * (nww * nwd) + j * nwd
    kwin = jax.lax.broadcasted_iota(jnp.int32, (ww, DC), 1) // (DC // nwd)
    vis = jnp.zeros((ww, DC), jnp.float32)
    for k in range(nwd):
        kv = keep_ref[b, base + k]
        vis = jnp.where(kwin == k, kv, vis)
    xb = x_ref[...].reshape(xm_ref.shape)
    xm_ref[...] = xb * vis[None, None]


def kernel(x, noise):
    B, H, W, D, C = x.shape
    wh, ww, wd = _WINDOW
    assert H % wh == 0 and W % ww == 0 and D % wd == 0
    nwh, nww, nwd = H // wh, W // ww, D // wd
    num_windows = nwh * nww * nwd
    len_keep = int(num_windows * (1 - _MASK_RATIO))

    nrow = noise.reshape(B, 1, num_windows)
    ncol = noise.reshape(B, num_windows, 1)
    keep = pl.pallas_call(
        functools.partial(
            _keep_kernel, num_windows=num_windows, len_keep=len_keep
        ),
        grid=(B,),
        in_specs=[
            pl.BlockSpec((1, 1, num_windows), lambda b: (b, 0, 0)),
            pl.BlockSpec((1, num_windows, 1), lambda b: (b, 0, 0)),
        ],
        out_specs=pl.BlockSpec((1, 1, num_windows), lambda b: (b, 0, 0)),
        out_shape=jax.ShapeDtypeStruct((B, 1, num_windows), jnp.float32),
    )(nrow, ncol)
    keep = keep.reshape(B, num_windows)

    DC = D * C
    x4 = x.reshape(B, H, W, DC)
    blk = pl.BlockSpec((1, wh, W, DC), lambda b, i: (b, i, 0, 0))
    out4 = jax.ShapeDtypeStruct((B, H, W, DC), x.dtype)
    params = pltpu.CompilerParams(
        dimension_semantics=("parallel", "parallel")
    )
    mask = pl.pallas_call(
        functools.partial(_maskout_kernel, nww=nww, nwd=nwd, W=W, DC=DC),
        grid=(B, nwh),
        in_specs=[pl.BlockSpec(memory_space=pltpu.SMEM)],
        out_specs=blk,
        out_shape=out4,
        compiler_params=params,
    )(keep)
    x_masked = pl.pallas_call(
        functools.partial(_xmask_kernel, nww=nww, nwd=nwd, ww=ww, DC=DC),
        grid=(B, nwh, nww),
        in_specs=[
            pl.BlockSpec(memory_space=pltpu.SMEM),
            pl.BlockSpec(
                (1, wh, ww, D, C), lambda b, i, j: (b, i, j, 0, 0)
            ),
        ],
        out_specs=pl.BlockSpec(
            (1, wh, ww, DC), lambda b, i, j: (b, i, j, 0)
        ),
        out_shape=out4,
        compiler_params=pltpu.CompilerParams(
            dimension_semantics=("parallel", "parallel", "parallel"),
        ),
    )(keep, x)
    shape5 = (B, H, W, D, C)
    return (x_masked.reshape(shape5), mask.reshape(shape5))


# speedup vs baseline: 1.4788x; 1.4788x over previous
import functools

import jax
import jax.numpy as jnp
from jax.experimental import pallas as pl
from jax.experimental.pallas import tpu as pltpu

_MASK_RATIO = 0.75
_WINDOW = (16, 16, 16)


def _keep_kernel(nrow_ref, ncol_ref, keep_ref, *, num_windows, len_keep):
    n = nrow_ref[0]
    nc = ncol_ref[0]
    wp = jax.lax.broadcasted_iota(jnp.int32, (num_windows, num_windows), 0)
    wo = jax.lax.broadcasted_iota(jnp.int32, (num_windows, num_windows), 1)
    m = (nc < n) | ((nc == n) & (wp < wo))
    rank = jnp.sum(m.astype(jnp.int32), axis=0, keepdims=True)
    keep_ref[0] = (rank < len_keep).astype(jnp.float32)


def _slab_vis(keep_ref, *, nww, nwd, W, DC):
    b = pl.program_id(0)
    i = pl.program_id(1)
    base = i * (nww * nwd)
    jwin = jax.lax.broadcasted_iota(jnp.int32, (W, DC), 0) // (W // nww)
    kwin = jax.lax.broadcasted_iota(jnp.int32, (W, DC), 1) // (DC // nwd)
    vis = jnp.zeros((W, DC), jnp.float32)
    for j in range(nww):
        for k in range(nwd):
            kv = keep_ref[b, base + j * nwd + k]
            vis = jnp.where((jwin == j) & (kwin == k), kv, vis)
    return vis


def _maskout_kernel(keep_ref, mask_ref, *, nww, nwd, W, DC):
    vis = _slab_vis(keep_ref, nww=nww, nwd=nwd, W=W, DC=DC)
    mask_ref[...] = jnp.broadcast_to(1.0 - vis[None, None], mask_ref.shape)


def _xmask_kernel(keep_ref, x0_ref, x1_ref, xm_ref, *, nww, nwd, W, DC):
    i = pl.program_id(0)
    b = pl.program_id(1)
    base = i * (nww * nwd)
    jwin = jax.lax.broadcasted_iota(jnp.int32, (W, DC), 0) // (W // nww)
    kwin = jax.lax.broadcasted_iota(jnp.int32, (W, DC), 1) // (DC // nwd)
    vis = jnp.zeros((W, DC), jnp.float32)
    for j in range(nww):
        for k in range(nwd):
            kv = keep_ref[b, base + j * nwd + k]
            vis = jnp.where((jwin == j) & (kwin == k), kv, vis)
    xb = jnp.where(b == 0, x0_ref[...], x1_ref[...])
    xm_ref[...] = xb * vis[None, None]


def kernel(x, noise):
    B, H, W, D, C = x.shape
    wh, ww, wd = _WINDOW
    assert H % wh == 0 and W % ww == 0 and D % wd == 0
    nwh, nww, nwd = H // wh, W // ww, D // wd
    num_windows = nwh * nww * nwd
    len_keep = int(num_windows * (1 - _MASK_RATIO))

    nrow = noise.reshape(B, 1, num_windows)
    ncol = noise.reshape(B, num_windows, 1)
    keep = pl.pallas_call(
        functools.partial(
            _keep_kernel, num_windows=num_windows, len_keep=len_keep
        ),
        grid=(B,),
        in_specs=[
            pl.BlockSpec((1, 1, num_windows), lambda b: (b, 0, 0)),
            pl.BlockSpec((1, num_windows, 1), lambda b: (b, 0, 0)),
        ],
        out_specs=pl.BlockSpec((1, 1, num_windows), lambda b: (b, 0, 0)),
        out_shape=jax.ShapeDtypeStruct((B, 1, num_windows), jnp.float32),
    )(nrow, ncol)
    keep = keep.reshape(B, num_windows)

    DC = D * C
    x4 = x.reshape(B, H, W, DC)
    blk = pl.BlockSpec((1, wh, W, DC), lambda b, i: (b, i, 0, 0))
    out4 = jax.ShapeDtypeStruct((B, H, W, DC), x.dtype)
    params = pltpu.CompilerParams(
        dimension_semantics=("parallel", "parallel")
    )
    mask = pl.pallas_call(
        functools.partial(_maskout_kernel, nww=nww, nwd=nwd, W=W, DC=DC),
        grid=(B, nwh),
        in_specs=[pl.BlockSpec(memory_space=pltpu.SMEM)],
        out_specs=blk,
        out_shape=out4,
        compiler_params=params,
    )(keep)
    assert B == 2
    x4_0 = x4[0:1]
    x4_1 = x4[1:2]
    xblk = pl.BlockSpec((1, wh, W, DC), lambda i, b: (0, i, 0, 0))
    x_masked = pl.pallas_call(
        functools.partial(_xmask_kernel, nww=nww, nwd=nwd, W=W, DC=DC),
        grid=(nwh, B),
        in_specs=[pl.BlockSpec(memory_space=pltpu.SMEM), xblk, xblk],
        out_specs=pl.BlockSpec((1, wh, W, DC), lambda i, b: (b, i, 0, 0)),
        out_shape=out4,
        compiler_params=params,
    )(keep, x4_0, x4_1)
    shape5 = (B, H, W, D, C)
    return (x_masked.reshape(shape5), mask.reshape(shape5))


# speedup vs baseline: 1.7435x; 1.1790x over previous
import functools

import jax
import jax.numpy as jnp
from jax.experimental import pallas as pl
from jax.experimental.pallas import tpu as pltpu

_MASK_RATIO = 0.75
_WINDOW = (16, 16, 16)


def _keep_kernel(nrow_ref, ncol_ref, keep_ref, *, num_windows, len_keep):
    n = nrow_ref[0]
    nc = ncol_ref[0]
    wp = jax.lax.broadcasted_iota(jnp.int32, (num_windows, num_windows), 0)
    wo = jax.lax.broadcasted_iota(jnp.int32, (num_windows, num_windows), 1)
    m = (nc < n) | ((nc == n) & (wp < wo))
    rank = jnp.sum(m.astype(jnp.int32), axis=0, keepdims=True)
    keep_ref[0] = (rank < len_keep).astype(jnp.float32)


def _slab_vis(keep_ref, *, nww, nwd, W, DC):
    b = pl.program_id(0)
    i = pl.program_id(1)
    base = i * (nww * nwd)
    jwin = jax.lax.broadcasted_iota(jnp.int32, (W, DC), 0) // (W // nww)
    kwin = jax.lax.broadcasted_iota(jnp.int32, (W, DC), 1) // (DC // nwd)
    vis = jnp.zeros((W, DC), jnp.float32)
    for j in range(nww):
        for k in range(nwd):
            kv = keep_ref[b, base + j * nwd + k]
            vis = jnp.where((jwin == j) & (kwin == k), kv, vis)
    return vis


def _maskout_kernel(keep_ref, mask_ref, *, nww, nwd, W, DC):
    vis = _slab_vis(keep_ref, nww=nww, nwd=nwd, W=W, DC=DC)
    mask_ref[...] = jnp.broadcast_to(1.0 - vis[None, None], mask_ref.shape)


def _xmask_kernel(keep_ref, x_ref, xm_ref, *, nww, nwd, W, DC):
    vis = _slab_vis(keep_ref, nww=nww, nwd=nwd, W=W, DC=DC)
    xm_ref[...] = x_ref[...] * vis[None, None]


def kernel(x, noise):
    B, H, W, D, C = x.shape
    wh, ww, wd = _WINDOW
    assert H % wh == 0 and W % ww == 0 and D % wd == 0
    nwh, nww, nwd = H // wh, W // ww, D // wd
    num_windows = nwh * nww * nwd
    len_keep = int(num_windows * (1 - _MASK_RATIO))

    nrow = noise.reshape(B, 1, num_windows)
    ncol = noise.reshape(B, num_windows, 1)
    keep = pl.pallas_call(
        functools.partial(
            _keep_kernel, num_windows=num_windows, len_keep=len_keep
        ),
        grid=(B,),
        in_specs=[
            pl.BlockSpec((1, 1, num_windows), lambda b: (b, 0, 0)),
            pl.BlockSpec((1, num_windows, 1), lambda b: (b, 0, 0)),
        ],
        out_specs=pl.BlockSpec((1, 1, num_windows), lambda b: (b, 0, 0)),
        out_shape=jax.ShapeDtypeStruct((B, 1, num_windows), jnp.float32),
    )(nrow, ncol)
    keep = keep.reshape(B, num_windows)

    DC = D * C
    x4 = x.reshape(B, H, W, DC)
    blk = pl.BlockSpec((1, wh, W, DC), lambda b, i: (b, i, 0, 0))
    out4 = jax.ShapeDtypeStruct((B, H, W, DC), x.dtype)
    params = pltpu.CompilerParams(
        dimension_semantics=("parallel", "parallel")
    )
    mask = pl.pallas_call(
        functools.partial(_maskout_kernel, nww=nww, nwd=nwd, W=W, DC=DC),
        grid=(B, nwh),
        in_specs=[pl.BlockSpec(memory_space=pltpu.SMEM)],
        out_specs=blk,
        out_shape=out4,
        compiler_params=params,
    )(keep)
    x_masked = pl.pallas_call(
        functools.partial(_xmask_kernel, nww=nww, nwd=nwd, W=W, DC=DC),
        grid=(B, nwh),
        in_specs=[pl.BlockSpec(memory_space=pltpu.SMEM), blk],
        out_specs=blk,
        out_shape=out4,
        compiler_params=params,
    )(keep, x4)
    shape5 = (B, H, W, D, C)
    return (x_masked.reshape(shape5), mask.reshape(shape5))
